# baseline (speedup 1.0000x reference)
import functools

import jax
import jax.numpy as jnp
from jax import lax
from jax.experimental import pallas as pl
from jax.experimental.pallas import tpu as pltpu
from jax.experimental.pallas import tpu_sc as plsc

B = 4096
DN = 13
F = 26
V = 100000
D = 32

NC = 2
NS = 16
NW = NC * NS

VSP = 1536
NSP = V // VSP
VT = 128
VREM = 32
KPF = NSP + 1
NTASK = F * KPF

R = (B * F) // NW
CH = 128
C = R // CH
BSLOT = 4


def _sc_transpose(tab_t, tail_t):
    mesh = plsc.VectorSubcoreMesh(core_axis_name="c", subcore_axis_name="s")

    @functools.partial(
        pl.kernel,
        out_type=jax.ShapeDtypeStruct((F * V // 4, 128), jnp.float32),
        mesh=mesh,
        scratch_types=[
            pltpu.VMEM((D, VSP), jnp.float32),
            pltpu.VMEM((VSP // 4, 128), jnp.float32),
            pltpu.VMEM((D, VREM), jnp.float32),
        ],
        compiler_params=pltpu.CompilerParams(needs_layout_passes=False),
    )
    def k(tab_hbm, tail_hbm, out_hbm, blk_v, row_v, tail_v):
        wid = lax.axis_index("s") * NC + lax.axis_index("c")
        lane = lax.iota(jnp.int32, 16)

        def transpose(nrows):
            def tbody(r, _):
                for m in range(8):
                    dd = 16 * (m % 2) + lane
                    vv = jnp.full((16,), 4 * r + m // 2, jnp.int32)
                    val = plsc.load_gather(blk_v, [dd, vv])
                    row_v[r, pl.ds(m * 16, 16)] = val
                return 0

            lax.fori_loop(0, nrows, tbody, 0)

        def do_span(f, kk):
            v0 = pl.multiple_of(kk * VSP, VT)
            pltpu.sync_copy(tab_hbm.at[f, :, pl.ds(v0, VSP)], blk_v)
            transpose(VSP // 4)
            off = pl.multiple_of((f * V + kk * VSP) // 4, 8)
            pltpu.sync_copy(row_v, out_hbm.at[pl.ds(off, VSP // 4)])

        def do_last(f):
            v0 = NSP * VSP
            pltpu.sync_copy(tab_hbm.at[f, :, pl.ds(v0, VT)],
                            blk_v.at[:, pl.ds(0, VT)])
            transpose(VT // 4)
            off = pl.multiple_of((f * V + v0) // 4, 8)
            pltpu.sync_copy(row_v.at[pl.ds(0, VT // 4)],
                            out_hbm.at[pl.ds(off, VT // 4)])

        N_MY = (NTASK + NW - 1) // NW

        def outer(t, _):
            i = wid + t * NW
            i = jnp.where(i < NTASK, i, i - NTASK)
            f = i // KPF
            kk = i % KPF

            @pl.when(kk < NSP)
            def _():
                do_span(f, kk)

            @pl.when(kk == NSP)
            def _():
                do_last(f)
            return 0

        lax.fori_loop(0, N_MY, outer, 0)

        @pl.when(wid < F)
        def _():
            f = wid
            v0 = NSP * VSP + VT
            pltpu.sync_copy(tail_hbm.at[f], tail_v)

            def rbody(r, _):
                for m in range(8):
                    dd = 16 * (m % 2) + lane
                    vv = jnp.full((16,), 4 * r + m // 2, jnp.int32)
                    val = plsc.load_gather(tail_v, [dd, vv])
                    row_v[r, pl.ds(m * 16, 16)] = val
                return 0

            lax.fori_loop(0, VREM // 4, rbody, 0)
            roff = pl.multiple_of((f * V + v0) // 4, 8)
            pltpu.sync_copy(row_v.at[pl.ds(0, VREM // 4)],
                            out_hbm.at[pl.ds(roff, VREM // 4)])

    return k(tab_t, tail_t)


def _sc_gather(tab4, sparse_flat):
    mesh = plsc.VectorSubcoreMesh(core_axis_name="c", subcore_axis_name="s")

    @functools.partial(
        pl.kernel,
        out_type=jax.ShapeDtypeStruct((B * F // 4, 128), jnp.float32),
        mesh=mesh,
        scratch_types=[
            pltpu.VMEM((R,), jnp.int32),
            pltpu.VMEM((R,), jnp.int32),
            pltpu.VMEM((R,), jnp.int32),
            pltpu.VMEM((BSLOT, CH, 128), jnp.float32),
            pltpu.VMEM((BSLOT, CH // 4, 128), jnp.float32),
            [pltpu.SemaphoreType.DMA] * BSLOT,
            [pltpu.SemaphoreType.DMA] * BSLOT,
        ],
        compiler_params=pltpu.CompilerParams(needs_layout_passes=False),
    )
    def k(tab_hbm, sp_hbm, out_hbm, raw_v, rows_v, coff_v, ring_v, ost_v,
          gsems, wsems):
        wid = lax.axis_index("s") * NC + lax.axis_index("c")
        base = wid * R
        pltpu.sync_copy(sp_hbm.at[pl.ds(base, R)], raw_v)

        lane = lax.iota(jnp.int32, 16)

        def hash_body(i, _):
            s = raw_v[pl.ds(i * 16, 16)]
            pos = (base + i * 16) + lane
            g = (pos % F) * V + (s + 1) % V
            rows_v[pl.ds(i * 16, 16)] = lax.shift_right_logical(g, 2)
            coff_v[pl.ds(i * 16, 16)] = (g & 3) * 32
            return 0

        lax.fori_loop(0, R // 16, hash_body, 0)

        def fire(j, slot):
            pltpu.async_copy(tab_hbm.at[rows_v.at[pl.ds(j * CH, CH)]],
                             ring_v.at[slot], gsems[slot])

        def gwait(slot):
            pltpu.make_async_copy(tab_hbm.at[pl.ds(0, CH)],
                                  ring_v.at[slot], gsems[slot]).wait()

        def owait(slot):
            pltpu.make_async_copy(ost_v.at[slot],
                                  out_hbm.at[pl.ds(0, CH // 4)],
                                  wsems[slot]).wait()

        def extract(j, slot):
            def ebody(u, _):
                rb = u * 16 + lane
                cof = coff_v[pl.ds(j * CH + u * 16, 16)]
                orow = lax.shift_right_logical(rb, 2)
                ocol0 = (rb & 3) * 32
                for d in range(D):
                    val = plsc.load_gather(ring_v.at[slot], [rb, cof + d])
                    plsc.store_scatter(ost_v.at[slot], [orow, ocol0 + d], val)
                return 0

            lax.fori_loop(0, CH // 16, ebody, 0)

        def put(j, slot):
            off = pl.multiple_of(wid * (R // 4) + j * (CH // 4), 8)
            pltpu.async_copy(ost_v.at[slot],
                             out_hbm.at[pl.ds(off, CH // 4)], wsems[slot])

        for s0 in range(BSLOT - 1):
            fire(s0, s0)

        def outer(jj, _):
            for b in range(BSLOT):
                j = jj * BSLOT + b

                @pl.when(j + BSLOT - 1 < C)
                def _():
                    @pl.when(j >= 1)
                    def _():
                        owait((b + BSLOT - 1) % BSLOT)
                    fire(j + BSLOT - 1, (b + BSLOT - 1) % BSLOT)

                @pl.when(j < C)
                def _():
                    gwait(b)
                    extract(j, b)
                    put(j, b)
            return 0

        lax.fori_loop(0, (C + BSLOT - 1) // BSLOT, outer, 0)
        for tt in range(C - BSLOT, C):
            owait(tt % BSLOT)

    return k(tab4, sparse_flat)


BB = 512


def _mlp_body(dense_ref, emb_ref, mean_ref, std_ref, W1_ref, b1_ref, W2_ref,
              b2_ref, W3_ref, b3_ref, P1a_ref, P1b_ref, pb1_ref, P2_ref,
              pb2_ref, P3_ref, pb3_ref, out_ref):
    x = (dense_ref[...] - mean_ref[...]) / std_ref[...]
    h = jnp.maximum(jnp.dot(x, W1_ref[...], preferred_element_type=jnp.float32)
                    + b1_ref[...], 0.0)
    h = jnp.maximum(jnp.dot(h, W2_ref[...], preferred_element_type=jnp.float32)
                    + b2_ref[...], 0.0)
    dense_out = jnp.dot(h, W3_ref[...], preferred_element_type=jnp.float32) + b3_ref[...]
    h1 = jnp.dot(dense_out, P1a_ref[...], preferred_element_type=jnp.float32)
    h1 = h1 + jnp.dot(emb_ref[...], P1b_ref[...], preferred_element_type=jnp.float32)
    h1 = jnp.maximum(h1 + pb1_ref[...], 0.0)
    h2 = jnp.maximum(jnp.dot(h1, P2_ref[...], preferred_element_type=jnp.float32)
                     + pb2_ref[...], 0.0)
    logit = jnp.sum(h2 * P3_ref[...], axis=1) + pb3_ref[0, 0]
    out_ref[...] = jax.nn.sigmoid(logit)


def _tc_mlp(dense, emb, mean_r, std_r, W1, b1r, W2, b2r, W3, b3r, P1a, P1b,
            pb1r, P2, pb2r, P3r, pb3r):
    grid = (B // BB,)

    def full(shape):
        return pl.BlockSpec(shape, lambda i: (0, 0))

    return pl.pallas_call(
        _mlp_body,
        grid=grid,
        in_specs=[
            pl.BlockSpec((BB, DN), lambda i: (i, 0)),
            pl.BlockSpec((BB, F * D), lambda i: (i, 0)),
            full((1, DN)), full((1, DN)),
            full((DN, 512)), full((1, 512)),
            full((512, 256)), full((1, 256)),
            full((256, D)), full((1, D)),
            full((D, 512)), full((F * D, 512)), full((1, 512)),
            full((512, 256)), full((1, 256)),
            full((1, 256)), full((1, 1)),
        ],
        out_specs=pl.BlockSpec((BB,), lambda i: (i,)),
        out_shape=jax.ShapeDtypeStruct((B,), jnp.float32),
    )(dense, emb, mean_r, std_r, W1, b1r, W2, b2r, W3, b3r, P1a, P1b, pb1r,
      P2, pb2r, P3r, pb3r)


def kernel(dense_features, sparse_features, mean, std, W1, b1, W2, b2, W3, b3,
           tables, P1, pb1, P2, pb2, P3, pb3):
    tab_t = jnp.swapaxes(tables, 1, 2)
    tab4 = _sc_transpose(tab_t, tab_t[:, :, V - VREM:])
    emb4 = _sc_gather(tab4, sparse_features.reshape(B * F))
    emb = emb4.reshape(B, F * D)
    return _tc_mlp(dense_features, emb, mean.reshape(1, DN), std.reshape(1, DN),
                   W1, b1.reshape(1, 512), W2, b2.reshape(1, 256), W3,
                   b3.reshape(1, D), P1[:D], P1[D:], pb1.reshape(1, 512),
                   P2, pb2.reshape(1, 256), P3.reshape(1, 256),
                   pb3.reshape(1, 1))


# speedup vs baseline: 1.2896x; 1.2896x over previous
import functools

import jax
import jax.numpy as jnp
from jax import lax
from jax.experimental import pallas as pl
from jax.experimental.pallas import tpu as pltpu
from jax.experimental.pallas import tpu_sc as plsc

B = 4096
DN = 13
F = 26
V = 100000
D = 32

FH = F // 2
NC = 2
NS = 16
NW = NC * NS
R = (B * FH) // NW
CH = 128
C = R // CH


def _sc_gather(tables_flat, sparse_flat):
    mesh = plsc.VectorSubcoreMesh(core_axis_name="c", subcore_axis_name="s")

    @functools.partial(
        pl.kernel,
        out_type=jax.ShapeDtypeStruct((B * FH, D), jnp.float32),
        mesh=mesh,
        scratch_types=[
            pltpu.VMEM((R,), jnp.int32),
            pltpu.VMEM((C, CH), jnp.int32),
            pltpu.VMEM((R, D), jnp.float32),
            pltpu.SemaphoreType.DMA,
        ],
        compiler_params=pltpu.CompilerParams(use_tc_tiling_on_sc=False),
    )
    def k(tab_hbm, sp_hbm, out_hbm, raw_v, idx_v, rows_v, sem):
        wid = lax.axis_index("s") * NC + lax.axis_index("c")
        base = wid * R
        pltpu.sync_copy(sp_hbm.at[pl.ds(base, R)], raw_v)

        def hash_body(i, _):
            s = raw_v[pl.ds(i * 16, 16)]
            pos = (base + i * 16) + lax.iota(jnp.int32, 16)
            g = (pos % FH) * V + (s + 1) % V
            idx_v[i // 8, pl.ds((i % 8) * 16, 16)] = g
            return 0

        lax.fori_loop(0, R // 16, hash_body, 0)

        def fire(j, _):
            pltpu.async_copy(tab_hbm.at[idx_v.at[j]],
                             rows_v.at[pl.ds(j * CH, CH)], sem)
            return 0

        lax.fori_loop(0, C, fire, 0)
        pltpu.make_async_copy(tab_hbm.at[pl.ds(0, R)], rows_v, sem).wait()
        pltpu.sync_copy(rows_v, out_hbm.at[pl.ds(base, R)])

    return k(tables_flat, sparse_flat)


BB = 512
EH = FH * D


def _mlp_body(dense_ref, embA_ref, embB_ref, mean_ref, std_ref, W1_ref,
              b1_ref, W2_ref, b2_ref, W3_ref, b3_ref, P1a_ref, P1bA_ref,
              P1bB_ref, pb1_ref, P2_ref, pb2_ref, P3_ref, pb3_ref, out_ref):
    x = (dense_ref[...] - mean_ref[...]) / std_ref[...]
    h = jnp.maximum(jnp.dot(x, W1_ref[...], preferred_element_type=jnp.float32)
                    + b1_ref[...], 0.0)
    h = jnp.maximum(jnp.dot(h, W2_ref[...], preferred_element_type=jnp.float32)
                    + b2_ref[...], 0.0)
    dense_out = jnp.dot(h, W3_ref[...], preferred_element_type=jnp.float32) + b3_ref[...]
    h1 = jnp.dot(dense_out, P1a_ref[...], preferred_element_type=jnp.float32)
    h1 = h1 + jnp.dot(embA_ref[...], P1bA_ref[...],
                      preferred_element_type=jnp.float32)
    h1 = h1 + jnp.dot(embB_ref[...], P1bB_ref[...],
                      preferred_element_type=jnp.float32)
    h1 = jnp.maximum(h1 + pb1_ref[...], 0.0)
    h2 = jnp.maximum(jnp.dot(h1, P2_ref[...], preferred_element_type=jnp.float32)
                     + pb2_ref[...], 0.0)
    logit = jnp.sum(h2 * P3_ref[...], axis=1) + pb3_ref[0, 0]
    out_ref[...] = jax.nn.sigmoid(logit)


def _tc_mlp(dense, embA, embB, mean_r, std_r, W1, b1r, W2, b2r, W3, b3r,
            P1a, P1bA, P1bB, pb1r, P2, pb2r, P3r, pb3r):
    grid = (B // BB,)

    def full(shape):
        return pl.BlockSpec(shape, lambda i: (0, 0))

    return pl.pallas_call(
        _mlp_body,
        grid=grid,
        in_specs=[
            pl.BlockSpec((BB, DN), lambda i: (i, 0)),
            pl.BlockSpec((BB, EH), lambda i: (i, 0)),
            pl.BlockSpec((BB, EH), lambda i: (i, 0)),
            full((1, DN)), full((1, DN)),
            full((DN, 512)), full((1, 512)),
            full((512, 256)), full((1, 256)),
            full((256, D)), full((1, D)),
            full((D, 512)), full((EH, 512)), full((EH, 512)), full((1, 512)),
            full((512, 256)), full((1, 256)),
            full((1, 256)), full((1, 1)),
        ],
        out_specs=pl.BlockSpec((BB,), lambda i: (i,)),
        out_shape=jax.ShapeDtypeStruct((B,), jnp.float32),
    )(dense, embA, embB, mean_r, std_r, W1, b1r, W2, b2r, W3, b3r, P1a,
      P1bA, P1bB, pb1r, P2, pb2r, P3r, pb3r)


def kernel(dense_features, sparse_features, mean, std, W1, b1, W2, b2, W3, b3,
           tables, P1, pb1, P2, pb2, P3, pb3):
    embA = _sc_gather(tables[:FH].reshape(FH * V, D),
                      sparse_features[:, :FH].reshape(B * FH))
    embB = _sc_gather(tables[FH:].reshape(FH * V, D),
                      sparse_features[:, FH:].reshape(B * FH))
    return _tc_mlp(dense_features, embA.reshape(B, EH), embB.reshape(B, EH),
                   mean.reshape(1, DN), std.reshape(1, DN),
                   W1, b1.reshape(1, 512), W2, b2.reshape(1, 256), W3,
                   b3.reshape(1, D), P1[:D], P1[D:D + EH], P1[D + EH:],
                   pb1.reshape(1, 512), P2, pb2.reshape(1, 256),
                   P3.reshape(1, 256), pb3.reshape(1, 1))


# speedup vs baseline: 1.9771x; 1.5331x over previous
import functools

import jax
import jax.numpy as jnp
from jax import lax
from jax.experimental import pallas as pl
from jax.experimental.pallas import tpu as pltpu
from jax.experimental.pallas import tpu_sc as plsc

B = 4096
DN = 13
F = 26
V = 100000
D = 32

NC = 2
NS = 16
NW = NC * NS
R = (B * F) // NW
CH = 128
C = R // CH


def _sc_gather(tables_flat, sparse_flat):
    mesh = plsc.VectorSubcoreMesh(core_axis_name="c", subcore_axis_name="s")

    @functools.partial(
        pl.kernel,
        out_type=jax.ShapeDtypeStruct((B * F, D), jnp.float32),
        mesh=mesh,
        scratch_types=[
            pltpu.VMEM((R,), jnp.int32),
            pltpu.VMEM((C, CH), jnp.int32),
            pltpu.VMEM((R, D), jnp.float32),
            pltpu.SemaphoreType.DMA,
        ],
        compiler_params=pltpu.CompilerParams(use_tc_tiling_on_sc=False),
    )
    def k(tab_hbm, sp_hbm, out_hbm, raw_v, idx_v, rows_v, sem):
        wid = lax.axis_index("s") * NC + lax.axis_index("c")
        base = wid * R
        pltpu.sync_copy(sp_hbm.at[pl.ds(base, R)], raw_v)

        def hash_body(i, _):
            s = raw_v[pl.ds(i * 16, 16)]
            pos = (base + i * 16) + lax.iota(jnp.int32, 16)
            g = (pos % F) * V + (s + 1) % V
            idx_v[i // 8, pl.ds((i % 8) * 16, 16)] = g
            return 0

        lax.fori_loop(0, R // 16, hash_body, 0)

        def fire(j, _):
            pltpu.async_copy(tab_hbm.at[idx_v.at[j]],
                             rows_v.at[pl.ds(j * CH, CH)], sem)
            return 0

        lax.fori_loop(0, C, fire, 0)
        pltpu.make_async_copy(tab_hbm.at[pl.ds(0, R)], rows_v, sem).wait()
        pltpu.sync_copy(rows_v, out_hbm.at[pl.ds(base, R)])

    return k(tables_flat, sparse_flat)


BB = 512


def _mlp_body(dense_ref, emb_ref, mean_ref, std_ref, W1_ref, b1_ref, W2_ref,
              b2_ref, W3_ref, b3_ref, P1a_ref, P1b_ref, pb1_ref, P2_ref,
              pb2_ref, P3_ref, pb3_ref, out_ref):
    x = (dense_ref[...] - mean_ref[...]) / std_ref[...]
    h = jnp.maximum(jnp.dot(x, W1_ref[...], preferred_element_type=jnp.float32)
                    + b1_ref[...], 0.0)
    h = jnp.maximum(jnp.dot(h, W2_ref[...], preferred_element_type=jnp.float32)
                    + b2_ref[...], 0.0)
    dense_out = jnp.dot(h, W3_ref[...], preferred_element_type=jnp.float32) + b3_ref[...]
    h1 = jnp.dot(dense_out, P1a_ref[...], preferred_element_type=jnp.float32)
    h1 = h1 + jnp.dot(emb_ref[...], P1b_ref[...], preferred_element_type=jnp.float32)
    h1 = jnp.maximum(h1 + pb1_ref[...], 0.0)
    h2 = jnp.maximum(jnp.dot(h1, P2_ref[...], preferred_element_type=jnp.float32)
                     + pb2_ref[...], 0.0)
    logit = jnp.sum(h2 * P3_ref[...], axis=1) + pb3_ref[0, 0]
    out_ref[...] = jax.nn.sigmoid(logit)


def _tc_mlp(dense, emb, mean_r, std_r, W1, b1r, W2, b2r, W3, b3r, P1a, P1b,
            pb1r, P2, pb2r, P3r, pb3r):
    grid = (B // BB,)

    def full(shape):
        return pl.BlockSpec(shape, lambda i: (0, 0))

    return pl.pallas_call(
        _mlp_body,
        grid=grid,
        in_specs=[
            pl.BlockSpec((BB, DN), lambda i: (i, 0)),
            pl.BlockSpec((BB, F * D), lambda i: (i, 0)),
            full((1, DN)), full((1, DN)),
            full((DN, 512)), full((1, 512)),
            full((512, 256)), full((1, 256)),
            full((256, D)), full((1, D)),
            full((D, 512)), full((F * D, 512)), full((1, 512)),
            full((512, 256)), full((1, 256)),
            full((1, 256)), full((1, 1)),
        ],
        out_specs=pl.BlockSpec((BB,), lambda i: (i,)),
        out_shape=jax.ShapeDtypeStruct((B,), jnp.float32),
    )(dense, emb, mean_r, std_r, W1, b1r, W2, b2r, W3, b3r, P1a, P1b, pb1r,
      P2, pb2r, P3r, pb3r)


def kernel(dense_features, sparse_features, mean, std, W1, b1, W2, b2, W3, b3,
           tables, P1, pb1, P2, pb2, P3, pb3):
    emb_flat = _sc_gather(tables.reshape(F * V, D),
                          sparse_features.reshape(B * F))
    emb = emb_flat.reshape(B, F * D)
    return _tc_mlp(dense_features, emb, mean.reshape(1, DN), std.reshape(1, DN),
                   W1, b1.reshape(1, 512), W2, b2.reshape(1, 256), W3,
                   b3.reshape(1, D), P1[:D], P1[D:], pb1.reshape(1, 512),
                   P2, pb2.reshape(1, 256), P3.reshape(1, 256),
                   pb3.reshape(1, 1))
